# baseline (speedup 1.0000x reference)
import functools

import jax
import jax.numpy as jnp
from jax import lax
from jax.experimental import pallas as pl
from jax.experimental.pallas import tpu as pltpu
from jax.experimental.pallas import tpu_sc as plsc

_NUM_EMBEDDINGS = 1_000_000
_DIM = 64
_BATCH = 4096 * 50
_NUM_WORKERS = 32
_B_PER_W = _BATCH // _NUM_WORKERS
_CHUNK = 128
_N_CHUNKS = _B_PER_W // _CHUNK


@jax.jit
def _embed(idx, table):
    mesh = plsc.VectorSubcoreMesh(core_axis_name="c", subcore_axis_name="s")

    @functools.partial(
        pl.kernel,
        mesh=mesh,
        out_type=jax.ShapeDtypeStruct((_BATCH, _DIM), jnp.float32),
        compiler_params=pltpu.CompilerParams(use_tc_tiling_on_sc=False),
        scratch_types=[
            pltpu.VMEM((_N_CHUNKS, _CHUNK), jnp.int32),
            pltpu.VMEM((2, _CHUNK, _DIM), jnp.float32),
            pltpu.SemaphoreType.DMA,
        ],
    )
    def emb(idx_hbm, table_hbm, out_hbm, idx_v, rows_v, gsem):
        wid = lax.axis_index("s") * 2 + lax.axis_index("c")
        base = wid * _B_PER_W
        pltpu.sync_copy(idx_hbm.at[wid], idx_v)

        pltpu.async_copy(table_hbm.at[idx_v.at[0]], rows_v.at[0], gsem)

        def body(g, _):
            buf = lax.rem(g, 2)
            nxt = lax.rem(g + 1, 2)

            @pl.when(g + 1 < _N_CHUNKS)
            def _():
                pltpu.async_copy(
                    table_hbm.at[idx_v.at[g + 1]], rows_v.at[nxt], gsem
                )

            pltpu.make_async_copy(
                table_hbm.at[idx_v.at[g]], rows_v.at[buf], gsem
            ).wait()
            pltpu.sync_copy(
                rows_v.at[buf],
                out_hbm.at[pl.ds(base + g * _CHUNK, _CHUNK)],
            )
            return 0

        lax.fori_loop(0, _N_CHUNKS, body, 0)

    return emb(idx, table)


def kernel(token_ids, W):
    idx = token_ids.reshape(_NUM_WORKERS, _N_CHUNKS, _CHUNK).astype(jnp.int32)
    out = _embed(idx, W)
    return out.reshape(token_ids.shape + (_DIM,))
